# baseline (speedup 1.0000x reference)
import functools

import jax
import jax.numpy as jnp
from jax import lax
from jax.experimental import pallas as pl
from jax.experimental.pallas import tpu as pltpu
from jax.experimental.pallas import tpu_sc as plsc

EMBED_DIM = 32
E_BLK = EMBED_DIM // 8
NUM_CORES = 2
NUM_SUBCORES = 16
NUM_WORKERS = NUM_CORES * NUM_SUBCORES
B_BLK = 128
T_CHUNK = 10
LANES = 16


@functools.lru_cache(maxsize=None)
def _make_gather(n_b, n_t):
    n_chunks = n_t // T_CHUNK
    assert n_b == NUM_WORKERS * B_BLK
    assert n_t % T_CHUNK == 0 and n_chunks % 2 == 0
    mesh = plsc.VectorSubcoreMesh(core_axis_name="c", subcore_axis_name="s")

    scratch = (
        [pltpu.VMEM((T_CHUNK * B_BLK,), jnp.int32) for _ in range(2)]
        + [pltpu.VMEM((T_CHUNK * B_BLK, EMBED_DIM), jnp.float32)
           for _ in range(2)]
        + [pltpu.VMEM((T_CHUNK, E_BLK, 8, B_BLK), jnp.float32)]
        + [pltpu.SemaphoreType.DMA for _ in range(3)]
    )

    @functools.partial(
        pl.kernel,
        mesh=mesh,
        compiler_params=pltpu.CompilerParams(use_tc_tiling_on_sc=False,
                                             needs_layout_passes=False,
                                             disable_bounds_checks=True),
        out_type=jax.ShapeDtypeStruct((n_t, E_BLK, NUM_WORKERS, 8, B_BLK),
                                      jnp.float32),
        scratch_types=scratch,
    )
    def gather_kernel(xt_hbm, table_hbm, out_hbm, idx0, idx1, rows0, rows1,
                      tiles_v, gsem0, gsem1, wsem):
        idx_bufs = (idx0, idx1)
        row_bufs = (rows0, rows1)
        gsems = (gsem0, gsem1)
        wid = lax.axis_index("s") * NUM_CORES + lax.axis_index("c")
        b0 = wid * B_BLK
        lane = lax.iota(jnp.int32, LANES)
        cols = [jnp.full((LANES,), e, jnp.int32) for e in range(EMBED_DIM)]

        def stage_and_fire(chunk, p):
            t0 = chunk * T_CHUNK
            for tj in range(T_CHUNK):
                pltpu.sync_copy(xt_hbm.at[t0 + tj, pl.ds(b0, B_BLK)],
                                idx_bufs[p].at[pl.ds(tj * B_BLK, B_BLK)])
            pltpu.async_copy(table_hbm.at[idx_bufs[p]], row_bufs[p], gsems[p])

        def gather_wait(p):
            pltpu.make_async_copy(table_hbm.at[idx_bufs[p]], row_bufs[p],
                                  gsems[p]).wait()

        def wb_wait():
            pltpu.make_async_copy(tiles_v,
                                  out_hbm.at[pl.ds(0, T_CHUNK), :, wid],
                                  wsem).wait()

        def transpose_and_wb(chunk, p):
            t0 = chunk * T_CHUNK
            rows_v = row_bufs[p]

            @pl.loop(0, T_CHUNK)
            def _(tj):
                for g in range(B_BLK // LANES):
                    row_idx = lane + (tj * B_BLK + g * LANES)
                    for e in range(EMBED_DIM):
                        vals = plsc.load_gather(rows_v, [row_idx, cols[e]])
                        tiles_v[tj, e // 8, e % 8,
                                pl.ds(g * LANES, LANES)] = vals
            pltpu.async_copy(tiles_v,
                             out_hbm.at[pl.ds(t0, T_CHUNK), :, wid], wsem)

        stage_and_fire(0, 0)

        @pl.loop(0, n_chunks // 2)
        def _(grp):
            for p in range(2):
                c = grp * 2 + p
                @pl.when(c + 1 < n_chunks)
                def _():
                    stage_and_fire(c + 1, 1 - p)
                gather_wait(p)
                @pl.when(c > 0)
                def _():
                    wb_wait()
                transpose_and_wb(c, p)

        wb_wait()

    return gather_kernel


def kernel(x, tool_embeddings):
    n_b, n_t = x.shape
    v5 = _make_gather(n_b, n_t)(x.T, tool_embeddings)
    return v5.transpose(2, 4, 0, 1, 3).reshape(n_b, n_t, EMBED_DIM)


# speedup vs baseline: 1.2337x; 1.2337x over previous
import functools

import jax
import jax.numpy as jnp
from jax import lax
from jax.experimental import pallas as pl
from jax.experimental.pallas import tpu as pltpu
from jax.experimental.pallas import tpu_sc as plsc

EMBED_DIM = 32
NUM_CORES = 2
NUM_SUBCORES = 16
NUM_WORKERS = NUM_CORES * NUM_SUBCORES

CHUNK = 2560


@functools.lru_cache(maxsize=None)
def _make_gather(n_idx):
    b_per_w = n_idx // NUM_WORKERS
    n_chunks = b_per_w // CHUNK
    assert n_idx % NUM_WORKERS == 0 and b_per_w % CHUNK == 0
    mesh = plsc.VectorSubcoreMesh(core_axis_name="c", subcore_axis_name="s")

    @functools.partial(
        pl.kernel,
        mesh=mesh,
        compiler_params=pltpu.CompilerParams(use_tc_tiling_on_sc=False),
        out_type=jax.ShapeDtypeStruct((n_idx, EMBED_DIM), jnp.float32),
        scratch_types=[
            pltpu.VMEM((CHUNK,), jnp.int32),
            pltpu.VMEM((CHUNK, EMBED_DIM), jnp.float32),
            pltpu.SemaphoreType.DMA,
        ],
    )
    def gather_kernel(idx_hbm, table_hbm, out_hbm, idx_v, rows_v, sem):
        wid = lax.axis_index("s") * NUM_CORES + lax.axis_index("c")
        base = wid * b_per_w

        def body(i, carry):
            off = base + i * CHUNK
            pltpu.sync_copy(idx_hbm.at[pl.ds(off, CHUNK)], idx_v)
            pltpu.async_copy(table_hbm.at[idx_v], rows_v, sem).wait()
            pltpu.sync_copy(rows_v, out_hbm.at[pl.ds(off, CHUNK)])
            return carry

        lax.fori_loop(0, n_chunks, body, 0)

    return gather_kernel


def kernel(x, tool_embeddings):
    idx = x.reshape(-1)
    out = _make_gather(idx.shape[0])(idx, tool_embeddings)
    return out.reshape(x.shape + (EMBED_DIM,))
